# Initial kernel scaffold; baseline (speedup 1.0000x reference)
import jax
import jax.numpy as jnp
from jax.experimental import pallas as pl


def kernel(query, reference_points, key_padding_mask, cam_intrinsics, cam_extrinsics, mlvl_feats, Wq, bq, Wv, bv, Wo, bo):
    raise NotImplementedError("write your pallas kernel here")


# speedup vs baseline: 4.6710x; 4.6710x over previous
import functools

import jax
import jax.numpy as jnp
from jax import lax
from jax.experimental import pallas as pl
from jax.experimental.pallas import tpu as pltpu
from jax.experimental.pallas import tpu_sc as plsc

B, Q, D, N_CAM, L, HF, WF = 2, 8192, 256, 6, 2, 56, 100
HW = HF * WF
HWP = 5632
G = B * Q
MIN_R, MAX_R = -51.2, 51.2
ORIG_W, ORIG_H = 800.0, 448.0

_PREC = lax.Precision.HIGHEST


def _table_kernel(f0_ref, f1_ref, wv_ref, out_ref):
    m = (f0_ref[0] + f1_ref[0]) * 0.5
    wv = wv_ref[...]
    out_ref[0] = lax.dot_general(m, wv, (((0,), (1,)), ((), ())),
                                 preferred_element_type=jnp.float32,
                                 precision=_PREC)


def _build_tables(f0, f1, wv):
    hw_t = 512
    grid = (B * N_CAM, HWP // hw_t)
    return pl.pallas_call(
        _table_kernel,
        grid=grid,
        in_specs=[
            pl.BlockSpec((1, D, hw_t), lambda bc, t: (bc, 0, t)),
            pl.BlockSpec((1, D, hw_t), lambda bc, t: (bc, 0, t)),
            pl.BlockSpec((D, D), lambda bc, t: (0, 0)),
        ],
        out_specs=pl.BlockSpec((1, hw_t, D), lambda bc, t: (bc, t, 0)),
        out_shape=jax.ShapeDtypeStruct((B * N_CAM, HWP, D), jnp.float32),
    )(f0, f1, wv)


def _proj_kernel(rp_ref, inv_ref, intr_ref, idx_ref, w_ref):
    b = pl.program_id(0)
    x = rp_ref[0, 0] * (MAX_R - MIN_R) + MIN_R
    y = rp_ref[0, 1] * (MAX_R - MIN_R) + MIN_R
    z = rp_ref[0, 2] * (MAX_R - MIN_R) + MIN_R
    for cam in range(N_CAM):
        def e(j, i):
            return inv_ref[b, cam, 4 * j + i]
        pc0 = e(0, 0) * x + e(0, 1) * y + e(0, 2) * z + e(0, 3)
        pc1 = e(1, 0) * x + e(1, 1) * y + e(1, 2) * z + e(1, 3)
        pc2 = e(2, 0) * x + e(2, 1) * y + e(2, 2) * z + e(2, 3)
        invalid = pc2 < 1.5
        dsafe = jnp.maximum(pc2, 1.5)
        px = pc0 / dsafe
        py = pc1 / dsafe
        pz = pc2 / dsafe

        def it(j, i):
            return intr_ref[b, cam, 3 * j + i]
        pi0 = jnp.clip(it(0, 0) * px + it(0, 1) * py + it(0, 2) * pz,
                       -3000.0, 3000.0)
        pi1 = jnp.clip(it(1, 0) * px + it(1, 1) * py + it(1, 2) * pz,
                       -3000.0, 3000.0)
        gx = jnp.clip((pi0 * (WF / ORIG_W)) / (WF - 1.0) * 2.0 - 1.0,
                      -10.0, 10.0)
        gy = jnp.clip((pi1 * (HF / ORIG_H)) / (HF - 1.0) * 2.0 - 1.0,
                      -10.0, 10.0)
        gx = jnp.where(invalid, -100.0, gx)
        gy = jnp.where(invalid, -100.0, gy)
        sx = (gx + 1.0) * 0.5 * (WF - 1)
        sy = (gy + 1.0) * 0.5 * (HF - 1)
        x0 = jnp.floor(sx)
        y0 = jnp.floor(sy)
        x1 = x0 + 1.0
        y1 = y0 + 1.0
        wx1 = sx - x0
        wx0 = 1.0 - wx1
        wy1 = sy - y0
        wy0 = 1.0 - wy1
        vx0 = ((x0 >= 0.0) & (x0 <= WF - 1.0)).astype(jnp.float32)
        vx1 = ((x1 >= 0.0) & (x1 <= WF - 1.0)).astype(jnp.float32)
        vy0 = ((y0 >= 0.0) & (y0 <= HF - 1.0)).astype(jnp.float32)
        vy1 = ((y1 >= 0.0) & (y1 <= HF - 1.0)).astype(jnp.float32)
        ix0 = jnp.clip(x0, 0.0, WF - 1.0).astype(jnp.int32)
        ix1 = jnp.clip(x1, 0.0, WF - 1.0).astype(jnp.int32)
        iy0 = jnp.clip(y0, 0.0, HF - 1.0).astype(jnp.int32)
        iy1 = jnp.clip(y1, 0.0, HF - 1.0).astype(jnp.int32)
        base = (b * N_CAM + cam) * HWP
        idx_ref[0, 4 * cam + 0] = base + iy0 * WF + ix0
        idx_ref[0, 4 * cam + 1] = base + iy0 * WF + ix1
        idx_ref[0, 4 * cam + 2] = base + iy1 * WF + ix0
        idx_ref[0, 4 * cam + 3] = base + iy1 * WF + ix1
        w_ref[0, 4 * cam + 0] = wx0 * wy0 * vx0 * vy0
        w_ref[0, 4 * cam + 1] = wx1 * wy0 * vx1 * vy0
        w_ref[0, 4 * cam + 2] = wx0 * wy1 * vx0 * vy1
        w_ref[0, 4 * cam + 3] = wx1 * wy1 * vx1 * vy1


def _project(rp_t, inv_flat, intr_flat):
    rows = Q // 128
    grid = (B, rows // 8)
    out_shapes = (
        jax.ShapeDtypeStruct((B, 4 * N_CAM, rows, 128), jnp.int32),
        jax.ShapeDtypeStruct((B, 4 * N_CAM, rows, 128), jnp.float32),
    )
    return pl.pallas_call(
        _proj_kernel,
        grid=grid,
        in_specs=[
            pl.BlockSpec((1, 3, 8, 128), lambda b, t: (b, 0, t, 0)),
            pl.BlockSpec(memory_space=pltpu.SMEM),
            pl.BlockSpec(memory_space=pltpu.SMEM),
        ],
        out_specs=(
            pl.BlockSpec((1, 4 * N_CAM, 8, 128), lambda b, t: (b, 0, t, 0)),
            pl.BlockSpec((1, 4 * N_CAM, 8, 128), lambda b, t: (b, 0, t, 0)),
        ),
        out_shape=out_shapes,
    )(rp_t, inv_flat, intr_flat)


_NW = 32
_QW = G // _NW
_CH = 64
_NCHUNK = _QW // _CH


def _sc_body(t_hbm, idx_hbm, w_hbm, out_hbm,
             idx_v, w_v, r0, r1, r2, r3, mx, sem):
    cid = lax.axis_index("c")
    sid = lax.axis_index("s")
    wid = sid * 2 + cid

    def chunk_body(ci, _):
        qbase = wid * _QW + ci * _CH
        chunkid = wid * _NCHUNK + ci
        for cam in range(N_CAM):
            pltpu.sync_copy(idx_hbm.at[chunkid, cam], idx_v)
            pltpu.sync_copy(w_hbm.at[chunkid, cam], w_v)
            cps = [pltpu.async_copy(t_hbm.at[idx_v.at[c]], r, sem)
                   for c, r in enumerate((r0, r1, r2, r3))]
            for cp in cps:
                cp.wait()

            def q_body(i, _):
                w0 = w_v[0, i, :]
                w1 = w_v[1, i, :]
                w2 = w_v[2, i, :]
                w3 = w_v[3, i, :]
                for j in range(D // 16):
                    sl = pl.ds(16 * j, 16)
                    acc = (w0 * r0[i, sl] + w1 * r1[i, sl]
                           + w2 * r2[i, sl] + w3 * r3[i, sl])
                    if cam > 0:
                        acc = jnp.maximum(acc, mx[i, sl])
                    mx[i, sl] = acc
                return 0

            lax.fori_loop(0, _CH, q_body, 0)
        pltpu.sync_copy(mx, out_hbm.at[pl.ds(qbase, _CH)])
        return 0

    lax.fori_loop(0, _NCHUNK, chunk_body, 0)


def _sc_sample(t_flat, idx_sc, w_sc):
    mesh = plsc.VectorSubcoreMesh(core_axis_name="c", subcore_axis_name="s")
    k = functools.partial(
        pl.kernel,
        mesh=mesh,
        out_type=jax.ShapeDtypeStruct((G, D), jnp.float32),
        scratch_types=[
            pltpu.VMEM((4, _CH), jnp.int32),
            pltpu.VMEM((4, _CH, 16), jnp.float32),
            pltpu.VMEM((_CH, D), jnp.float32),
            pltpu.VMEM((_CH, D), jnp.float32),
            pltpu.VMEM((_CH, D), jnp.float32),
            pltpu.VMEM((_CH, D), jnp.float32),
            pltpu.VMEM((_CH, D), jnp.float32),
            pltpu.SemaphoreType.DMA,
        ],
    )(_sc_body)
    return k(t_flat, idx_sc, w_sc)


def _fuse_kernel(q_ref, s_ref, wq_ref, wo_ref, bias_ref, out_ref):
    qb = q_ref[...]
    qp = lax.dot_general(qb, wq_ref[...], (((1,), (1,)), ((), ())),
                         preferred_element_type=jnp.float32,
                         precision=_PREC) + bias_ref[0]
    h = jax.nn.relu(qp + s_ref[...] + bias_ref[1]) + qp
    out_ref[...] = lax.dot_general(h, wo_ref[...], (((1,), (1,)), ((), ())),
                                   preferred_element_type=jnp.float32,
                                   precision=_PREC) + bias_ref[2]


def _fuse(qflat, s, wq, wo, biases):
    qt = 512
    return pl.pallas_call(
        _fuse_kernel,
        grid=(G // qt,),
        in_specs=[
            pl.BlockSpec((qt, D), lambda i: (i, 0)),
            pl.BlockSpec((qt, D), lambda i: (i, 0)),
            pl.BlockSpec((D, D), lambda i: (0, 0)),
            pl.BlockSpec((D, D), lambda i: (0, 0)),
            pl.BlockSpec((8, D), lambda i: (0, 0)),
        ],
        out_specs=pl.BlockSpec((qt, D), lambda i: (i, 0)),
        out_shape=jax.ShapeDtypeStruct((G, D), jnp.float32),
    )(qflat, s, wq, wo, biases)


def kernel(query, reference_points, key_padding_mask, cam_intrinsics,
           cam_extrinsics, mlvl_feats, Wq, bq, Wv, bv, Wo, bo):
    b, nq, d = query.shape

    f = mlvl_feats.reshape(L, B * N_CAM, D, HW)
    t_flat = _build_tables(f[0], f[1], Wv).reshape(B * N_CAM * HWP, D)

    inv_ext = jnp.linalg.inv(cam_extrinsics)
    inv_ext = jnp.nan_to_num(inv_ext, nan=0.0, posinf=1e6, neginf=-1e6)
    inv_flat = inv_ext.reshape(B, N_CAM, 16)
    intr_flat = cam_intrinsics.reshape(B, N_CAM, 9)
    rp_t = reference_points.transpose(0, 2, 1).reshape(B, 3, Q // 128, 128)
    idx4, w4 = _project(rp_t, inv_flat, intr_flat)
    nch = G // _CH
    idx_sc = (idx4.transpose(0, 2, 3, 1).reshape(nch, _CH, 4 * N_CAM)
              .transpose(0, 2, 1).reshape(nch, N_CAM, 4, _CH))
    w_sc = (w4.transpose(0, 2, 3, 1).reshape(nch, _CH, 4 * N_CAM)
            .transpose(0, 2, 1).reshape(nch, N_CAM, 4, _CH))
    w_exp = jnp.broadcast_to(w_sc[..., None], (nch, N_CAM, 4, _CH, 16))

    s = _sc_sample(t_flat, idx_sc, w_exp)

    biases = jnp.zeros((8, D), jnp.float32)
    biases = biases.at[0].set(bq).at[1].set(bv).at[2].set(bo)
    out = _fuse(query.reshape(G, D), s, Wq, Wo, biases)
    out = out.reshape(B, Q, D)
    return out * (~key_padding_mask)[..., None].astype(out.dtype)


# speedup vs baseline: 5.0518x; 1.0815x over previous
import functools

import jax
import jax.numpy as jnp
from jax import lax
from jax.experimental import pallas as pl
from jax.experimental.pallas import tpu as pltpu
from jax.experimental.pallas import tpu_sc as plsc

B, Q, D, N_CAM, L, HF, WF = 2, 8192, 256, 6, 2, 56, 100
HW = HF * WF
HWP = 5632
G = B * Q
MIN_R, MAX_R = -51.2, 51.2
ORIG_W, ORIG_H = 800.0, 448.0

_PREC = lax.Precision.HIGHEST


def _table_kernel(f0_ref, f1_ref, wv_ref, out_ref):
    m = (f0_ref[0] + f1_ref[0]) * 0.5
    wv = wv_ref[...]
    out_ref[0] = lax.dot_general(m, wv, (((0,), (1,)), ((), ())),
                                 preferred_element_type=jnp.float32,
                                 precision=_PREC)


def _build_tables(f0, f1, wv):
    hw_t = 512
    grid = (B * N_CAM, HWP // hw_t)
    return pl.pallas_call(
        _table_kernel,
        grid=grid,
        in_specs=[
            pl.BlockSpec((1, D, hw_t), lambda bc, t: (bc, 0, t)),
            pl.BlockSpec((1, D, hw_t), lambda bc, t: (bc, 0, t)),
            pl.BlockSpec((D, D), lambda bc, t: (0, 0)),
        ],
        out_specs=pl.BlockSpec((1, hw_t, D), lambda bc, t: (bc, t, 0)),
        out_shape=jax.ShapeDtypeStruct((B * N_CAM, HWP, D), jnp.float32),
    )(f0, f1, wv)


def _proj_kernel(rp_ref, inv_ref, intr_ref, idx_ref, w_ref):
    b = pl.program_id(0)
    x = rp_ref[0, 0] * (MAX_R - MIN_R) + MIN_R
    y = rp_ref[0, 1] * (MAX_R - MIN_R) + MIN_R
    z = rp_ref[0, 2] * (MAX_R - MIN_R) + MIN_R
    for cam in range(N_CAM):
        def e(j, i):
            return inv_ref[b, cam, 4 * j + i]
        pc0 = e(0, 0) * x + e(0, 1) * y + e(0, 2) * z + e(0, 3)
        pc1 = e(1, 0) * x + e(1, 1) * y + e(1, 2) * z + e(1, 3)
        pc2 = e(2, 0) * x + e(2, 1) * y + e(2, 2) * z + e(2, 3)
        invalid = pc2 < 1.5
        dsafe = jnp.maximum(pc2, 1.5)
        px = pc0 / dsafe
        py = pc1 / dsafe
        pz = pc2 / dsafe

        def it(j, i):
            return intr_ref[b, cam, 3 * j + i]
        pi0 = jnp.clip(it(0, 0) * px + it(0, 1) * py + it(0, 2) * pz,
                       -3000.0, 3000.0)
        pi1 = jnp.clip(it(1, 0) * px + it(1, 1) * py + it(1, 2) * pz,
                       -3000.0, 3000.0)
        gx = jnp.clip((pi0 * (WF / ORIG_W)) / (WF - 1.0) * 2.0 - 1.0,
                      -10.0, 10.0)
        gy = jnp.clip((pi1 * (HF / ORIG_H)) / (HF - 1.0) * 2.0 - 1.0,
                      -10.0, 10.0)
        gx = jnp.where(invalid, -100.0, gx)
        gy = jnp.where(invalid, -100.0, gy)
        sx = (gx + 1.0) * 0.5 * (WF - 1)
        sy = (gy + 1.0) * 0.5 * (HF - 1)
        x0 = jnp.floor(sx)
        y0 = jnp.floor(sy)
        x1 = x0 + 1.0
        y1 = y0 + 1.0
        wx1 = sx - x0
        wx0 = 1.0 - wx1
        wy1 = sy - y0
        wy0 = 1.0 - wy1
        vx0 = ((x0 >= 0.0) & (x0 <= WF - 1.0)).astype(jnp.float32)
        vx1 = ((x1 >= 0.0) & (x1 <= WF - 1.0)).astype(jnp.float32)
        vy0 = ((y0 >= 0.0) & (y0 <= HF - 1.0)).astype(jnp.float32)
        vy1 = ((y1 >= 0.0) & (y1 <= HF - 1.0)).astype(jnp.float32)
        ix0 = jnp.clip(x0, 0.0, WF - 1.0).astype(jnp.int32)
        ix1 = jnp.clip(x1, 0.0, WF - 1.0).astype(jnp.int32)
        iy0 = jnp.clip(y0, 0.0, HF - 1.0).astype(jnp.int32)
        iy1 = jnp.clip(y1, 0.0, HF - 1.0).astype(jnp.int32)
        base = (b * N_CAM + cam) * HWP
        idx_ref[0, 4 * cam + 0] = base + iy0 * WF + ix0
        idx_ref[0, 4 * cam + 1] = base + iy0 * WF + ix1
        idx_ref[0, 4 * cam + 2] = base + iy1 * WF + ix0
        idx_ref[0, 4 * cam + 3] = base + iy1 * WF + ix1
        w_ref[0, 4 * cam + 0] = wx0 * wy0 * vx0 * vy0
        w_ref[0, 4 * cam + 1] = wx1 * wy0 * vx1 * vy0
        w_ref[0, 4 * cam + 2] = wx0 * wy1 * vx0 * vy1
        w_ref[0, 4 * cam + 3] = wx1 * wy1 * vx1 * vy1


def _project(rp_t, inv_flat, intr_flat):
    rows = Q // 128
    grid = (B, rows // 8)
    out_shapes = (
        jax.ShapeDtypeStruct((B, 4 * N_CAM, rows, 128), jnp.int32),
        jax.ShapeDtypeStruct((B, 4 * N_CAM, rows, 128), jnp.float32),
    )
    return pl.pallas_call(
        _proj_kernel,
        grid=grid,
        in_specs=[
            pl.BlockSpec((1, 3, 8, 128), lambda b, t: (b, 0, t, 0)),
            pl.BlockSpec(memory_space=pltpu.SMEM),
            pl.BlockSpec(memory_space=pltpu.SMEM),
        ],
        out_specs=(
            pl.BlockSpec((1, 4 * N_CAM, 8, 128), lambda b, t: (b, 0, t, 0)),
            pl.BlockSpec((1, 4 * N_CAM, 8, 128), lambda b, t: (b, 0, t, 0)),
        ),
        out_shape=out_shapes,
    )(rp_t, inv_flat, intr_flat)


_NW = 32
_QW = G // _NW
_CH = 32
_NCHUNK = _QW // _CH


def _sc_body(t_hbm, idx_hbm, w_hbm, out_hbm,
             idx_v, wa, wb, ra0, ra1, ra2, ra3, rb0, rb1, rb2, rb3, mx,
             sem_a, sem_b, sem_o):
    cid = lax.axis_index("c")
    sid = lax.axis_index("s")
    wid = sid * 2 + cid
    rows = ((ra0, ra1, ra2, ra3), (rb0, rb1, rb2, rb3))
    wbufs = (wa, wb)
    sems = (sem_a, sem_b)

    def gathers(chunkid, cam, p):
        cps = [pltpu.async_copy(t_hbm.at[idx_v.at[cam, c]], rows[p][c],
                                sems[p])
               for c in range(4)]
        cps.append(pltpu.async_copy(w_hbm.at[chunkid, cam], wbufs[p],
                                    sems[p]))
        return cps

    def chunk_body(ci, _):
        qbase = wid * _QW + ci * _CH
        chunkid = wid * _NCHUNK + ci
        pltpu.sync_copy(idx_hbm.at[chunkid], idx_v)
        cur = gathers(chunkid, 0, 0)
        for cam in range(N_CAM):
            p = cam & 1
            nxt = (gathers(chunkid, cam + 1, p ^ 1)
                   if cam < N_CAM - 1 else [])
            for cp in cur:
                cp.wait()
            cur = nxt
            r0, r1, r2, r3 = rows[p]
            w_v = wbufs[p]

            def q_body(i, _):
                w0 = w_v[0, i, :]
                w1 = w_v[1, i, :]
                w2 = w_v[2, i, :]
                w3 = w_v[3, i, :]
                for j in range(D // 16):
                    sl = pl.ds(16 * j, 16)
                    acc = (w0 * r0[i, sl] + w1 * r1[i, sl]
                           + w2 * r2[i, sl] + w3 * r3[i, sl])
                    if cam > 0:
                        acc = jnp.maximum(acc, mx[i, sl])
                    mx[i, sl] = acc
                return 0

            lax.fori_loop(0, _CH, q_body, 0, unroll=2)
        pltpu.sync_copy(mx, out_hbm.at[pl.ds(qbase, _CH)])
        return 0

    lax.fori_loop(0, _NCHUNK, chunk_body, 0)


def _sc_sample(t_flat, idx_sc, w_sc):
    mesh = plsc.VectorSubcoreMesh(core_axis_name="c", subcore_axis_name="s")
    k = functools.partial(
        pl.kernel,
        mesh=mesh,
        out_type=jax.ShapeDtypeStruct((G, D), jnp.float32),
        scratch_types=[
            pltpu.VMEM((N_CAM, 4, _CH), jnp.int32),
            pltpu.VMEM((4, _CH, 16), jnp.float32),
            pltpu.VMEM((4, _CH, 16), jnp.float32),
            pltpu.VMEM((_CH, D), jnp.float32),
            pltpu.VMEM((_CH, D), jnp.float32),
            pltpu.VMEM((_CH, D), jnp.float32),
            pltpu.VMEM((_CH, D), jnp.float32),
            pltpu.VMEM((_CH, D), jnp.float32),
            pltpu.VMEM((_CH, D), jnp.float32),
            pltpu.VMEM((_CH, D), jnp.float32),
            pltpu.VMEM((_CH, D), jnp.float32),
            pltpu.VMEM((_CH, D), jnp.float32),
            pltpu.SemaphoreType.DMA,
            pltpu.SemaphoreType.DMA,
            pltpu.SemaphoreType.DMA,
        ],
    )(_sc_body)
    return k(t_flat, idx_sc, w_sc)


def _fuse_kernel(q_ref, s_ref, wq_ref, wo_ref, bias_ref, out_ref):
    qb = q_ref[...]
    qp = lax.dot_general(qb, wq_ref[...], (((1,), (1,)), ((), ())),
                         preferred_element_type=jnp.float32,
                         precision=_PREC) + bias_ref[0]
    h = jax.nn.relu(qp + s_ref[...] + bias_ref[1]) + qp
    out_ref[...] = lax.dot_general(h, wo_ref[...], (((1,), (1,)), ((), ())),
                                   preferred_element_type=jnp.float32,
                                   precision=_PREC) + bias_ref[2]


def _fuse(qflat, s, wq, wo, biases):
    qt = 512
    return pl.pallas_call(
        _fuse_kernel,
        grid=(G // qt,),
        in_specs=[
            pl.BlockSpec((qt, D), lambda i: (i, 0)),
            pl.BlockSpec((qt, D), lambda i: (i, 0)),
            pl.BlockSpec((D, D), lambda i: (0, 0)),
            pl.BlockSpec((D, D), lambda i: (0, 0)),
            pl.BlockSpec((8, D), lambda i: (0, 0)),
        ],
        out_specs=pl.BlockSpec((qt, D), lambda i: (i, 0)),
        out_shape=jax.ShapeDtypeStruct((G, D), jnp.float32),
    )(qflat, s, wq, wo, biases)


def kernel(query, reference_points, key_padding_mask, cam_intrinsics,
           cam_extrinsics, mlvl_feats, Wq, bq, Wv, bv, Wo, bo):
    b, nq, d = query.shape

    f = mlvl_feats.reshape(L, B * N_CAM, D, HW)
    t_flat = _build_tables(f[0], f[1], Wv).reshape(B * N_CAM * HWP, D)

    inv_ext = jnp.linalg.inv(cam_extrinsics)
    inv_ext = jnp.nan_to_num(inv_ext, nan=0.0, posinf=1e6, neginf=-1e6)
    inv_flat = inv_ext.reshape(B, N_CAM, 16)
    intr_flat = cam_intrinsics.reshape(B, N_CAM, 9)
    rp_t = reference_points.transpose(0, 2, 1).reshape(B, 3, Q // 128, 128)
    idx4, w4 = _project(rp_t, inv_flat, intr_flat)
    nch = G // _CH
    idx_sc = (idx4.transpose(0, 2, 3, 1).reshape(nch, _CH, 4 * N_CAM)
              .transpose(0, 2, 1).reshape(nch, N_CAM, 4, _CH))
    w_sc = (w4.transpose(0, 2, 3, 1).reshape(nch, _CH, 4 * N_CAM)
            .transpose(0, 2, 1).reshape(nch, N_CAM, 4, _CH))
    w_exp = jnp.broadcast_to(w_sc[..., None], (nch, N_CAM, 4, _CH, 16))

    s = _sc_sample(t_flat, idx_sc, w_exp)

    biases = jnp.zeros((8, D), jnp.float32)
    biases = biases.at[0].set(bq).at[1].set(bv).at[2].set(bo)
    out = _fuse(query.reshape(G, D), s, Wq, Wo, biases)
    out = out.reshape(B, Q, D)
    return out * (~key_padding_mask)[..., None].astype(out.dtype)
